# baseline (speedup 1.0000x reference)
import jax
import jax.numpy as jnp
from jax import lax
from jax.experimental import pallas as pl
from jax.experimental.pallas import tpu as pltpu

B, S, D = 32, 8, 768
E = 16
H = 2 * D
N = B * S
TB = 32

_F32 = jnp.float32
_I32 = jnp.int32


def _dot(a, b, dims):
    return lax.dot_general(a, b, (dims, ((), ())), preferred_element_type=_F32)


def _moe_kernel(x_ref, wg_ref, w1_hbm, w2_hbm, w3_hbm, out_ref,
                wb1, wb2, wb3, sem, meta_vm, meta_sm, sem2):
    def _copies(e, slot):
        qh = H // 4
        qd = D // 4
        cs = []
        for q in range(4):
            cs.append(pltpu.make_async_copy(
                w1_hbm.at[e, pl.ds(q * qh, qh)],
                wb1.at[slot, pl.ds(q * qh, qh)], sem.at[slot, q]))
            cs.append(pltpu.make_async_copy(
                w2_hbm.at[e, pl.ds(q * qh, qh)],
                wb2.at[slot, pl.ds(q * qh, qh)], sem.at[slot, 4 + q]))
            cs.append(pltpu.make_async_copy(
                w3_hbm.at[e, pl.ds(q * qd, qd)],
                wb3.at[slot, pl.ds(q * qd, qd)], sem.at[slot, 8 + q]))
        return cs

    def issue(e, slot):
        for c in _copies(e, slot):
            c.start()

    def wait(e, slot):
        for c in _copies(e, slot):
            c.wait()

    issue(0, 0)
    issue(1, 1)

    x = x_ref[...]
    wg = wg_ref[...]
    gate = _dot(x, wg, ((1,), (1,)))

    e_iota = lax.broadcasted_iota(_I32, (N, E), 1)
    mx = jnp.max(gate, axis=1, keepdims=True)
    eid = jnp.min(jnp.where(gate == mx, e_iota, E), axis=1, keepdims=True)
    oh = (e_iota == eid).astype(_F32)

    cnt = jnp.sum(oh, axis=0, keepdims=True)
    lt16 = (lax.broadcasted_iota(_I32, (E, E), 0)
            < lax.broadcasted_iota(_I32, (E, E), 1)).astype(_F32)
    off = _dot(cnt, lt16, ((1,), (0,)))
    le256 = (lax.broadcasted_iota(_I32, (N, N), 1)
             <= lax.broadcasted_iota(_I32, (N, N), 0)).astype(_F32)
    ranks = _dot(le256, oh, ((1,), (0,)))
    rank = jnp.sum(ranks * oh, axis=1, keepdims=True)
    off_tok = jnp.sum(off * oh, axis=1, keepdims=True)
    posv = (off_tok + rank - 1.0).astype(_I32)

    meta_vm[0:1, :] = cnt.astype(_I32)
    meta_vm[1:2, :] = off.astype(_I32)
    cp = pltpu.make_async_copy(meta_vm, meta_sm, sem2)
    cp.start()
    cp.wait()

    r_iota = lax.broadcasted_iota(_I32, (N, TB), 1)
    out_ref[...] = jnp.zeros_like(out_ref)

    for e in range(E):
        slot = e % 3
        if e + 2 < E:
            issue(e + 2, (e + 2) % 3)
        wait(e, slot)
        w1 = wb1[slot]
        w2 = wb2[slot]
        w3 = wb3[slot]
        cnt_e = meta_sm[0, e]
        off_e = meta_sm[1, e]
        nblk_e = (cnt_e + (TB - 1)) // TB
        limit = off_e + cnt_e

        def body(j, _):
            base = off_e + j * TB
            p2 = ((posv - base == r_iota) & (posv < limit)).astype(_F32)
            xblk = _dot(p2, x, ((0,), (0,)))
            h = _dot(xblk, w1, ((1,), (1,)))
            v = _dot(xblk, w2, ((1,), (1,)))
            act = h * jax.nn.sigmoid(h) * v
            y = _dot(act, w3, ((1,), (1,)))
            out_ref[...] += _dot(p2, y, ((1,), (0,)))
            return 0

        lax.fori_loop(0, nblk_e, body, 0)


def kernel(x, Wg, W1, W2, W3):
    x2 = x.reshape(N, D)
    out = pl.pallas_call(
        _moe_kernel,
        in_specs=[
            pl.BlockSpec(memory_space=pltpu.VMEM),
            pl.BlockSpec(memory_space=pltpu.VMEM),
            pl.BlockSpec(memory_space=pl.ANY),
            pl.BlockSpec(memory_space=pl.ANY),
            pl.BlockSpec(memory_space=pl.ANY),
        ],
        out_shape=jax.ShapeDtypeStruct((N, D), _F32),
        scratch_shapes=[
            pltpu.VMEM((3, H, D), _F32),
            pltpu.VMEM((3, H, D), _F32),
            pltpu.VMEM((3, D, H), _F32),
            pltpu.SemaphoreType.DMA((3, 12)),
            pltpu.VMEM((2, E), _I32),
            pltpu.SMEM((2, E), _I32),
            pltpu.SemaphoreType.DMA,
        ],
        compiler_params=pltpu.CompilerParams(
            vmem_limit_bytes=100 * 1024 * 1024,
        ),
    )(x2, Wg, W1, W2, W3)
    return out.reshape(B, S, D)


# speedup vs baseline: 1.0630x; 1.0630x over previous
import jax
import jax.numpy as jnp
from jax import lax
from jax.experimental import pallas as pl
from jax.experimental.pallas import tpu as pltpu

B, S, D = 32, 8, 768
E = 16
H = 2 * D
N = B * S
TB = 32

_F32 = jnp.float32
_I32 = jnp.int32


def _dot(a, b, dims):
    return lax.dot_general(a, b, (dims, ((), ())), preferred_element_type=_F32)


def _moe_kernel(x_ref, wg_ref, w1_hbm, w2_hbm, w3_hbm, out_ref,
                wb1, wb2, wb3, sem, meta_vm, meta_sm, sem2):
    def _copies(e, slot):
        hh = H // 2
        hd = D // 2
        return [
            pltpu.make_async_copy(w1_hbm.at[e, pl.ds(0, hh)],
                                  wb1.at[slot, pl.ds(0, hh)], sem.at[slot, 0]),
            pltpu.make_async_copy(w1_hbm.at[e, pl.ds(hh, hh)],
                                  wb1.at[slot, pl.ds(hh, hh)], sem.at[slot, 1]),
            pltpu.make_async_copy(w2_hbm.at[e, pl.ds(0, hh)],
                                  wb2.at[slot, pl.ds(0, hh)], sem.at[slot, 2]),
            pltpu.make_async_copy(w2_hbm.at[e, pl.ds(hh, hh)],
                                  wb2.at[slot, pl.ds(hh, hh)], sem.at[slot, 3]),
            pltpu.make_async_copy(w3_hbm.at[e, pl.ds(0, hd)],
                                  wb3.at[slot, pl.ds(0, hd)], sem.at[slot, 4]),
            pltpu.make_async_copy(w3_hbm.at[e, pl.ds(hd, hd)],
                                  wb3.at[slot, pl.ds(hd, hd)], sem.at[slot, 5]),
        ]

    def issue(e, slot):
        for c in _copies(e, slot):
            c.start()

    def wait(e, slot):
        for c in _copies(e, slot):
            c.wait()

    issue(0, 0)
    issue(1, 1)

    x = x_ref[...]
    wg = wg_ref[...]
    gate = _dot(x, wg, ((1,), (1,)))

    e_iota = lax.broadcasted_iota(_I32, (N, E), 1)
    mx = jnp.max(gate, axis=1, keepdims=True)
    eid = jnp.min(jnp.where(gate == mx, e_iota, E), axis=1, keepdims=True)
    oh = (e_iota == eid).astype(_F32)

    cnt = jnp.sum(oh, axis=0, keepdims=True)
    lt16 = (lax.broadcasted_iota(_I32, (E, E), 0)
            < lax.broadcasted_iota(_I32, (E, E), 1)).astype(_F32)
    off = _dot(cnt, lt16, ((1,), (0,)))
    le256 = (lax.broadcasted_iota(_I32, (N, N), 1)
             <= lax.broadcasted_iota(_I32, (N, N), 0)).astype(_F32)
    ranks = _dot(le256, oh, ((1,), (0,)))
    rank = jnp.sum(ranks * oh, axis=1, keepdims=True)
    off_tok = jnp.sum(off * oh, axis=1, keepdims=True)
    posv = (off_tok + rank - 1.0).astype(_I32)

    meta_vm[0:1, :] = cnt.astype(_I32)
    meta_vm[1:2, :] = off.astype(_I32)
    cp = pltpu.make_async_copy(meta_vm, meta_sm, sem2)
    cp.start()
    cp.wait()

    r_iota = lax.broadcasted_iota(_I32, (N, TB), 1)
    out_ref[...] = jnp.zeros_like(out_ref)

    for e in range(E):
        slot = e % 3
        if e + 2 < E:
            issue(e + 2, (e + 2) % 3)
        wait(e, slot)
        w1 = wb1[slot]
        w2 = wb2[slot]
        w3 = wb3[slot]
        cnt_e = meta_sm[0, e]
        off_e = meta_sm[1, e]
        nblk_e = (cnt_e + (TB - 1)) // TB
        limit = off_e + cnt_e

        def body(j, _):
            base = off_e + j * TB
            p2 = ((posv - base == r_iota) & (posv < limit)).astype(_F32)
            xblk = _dot(p2, x, ((0,), (0,)))
            h = _dot(xblk, w1, ((1,), (1,)))
            v = _dot(xblk, w2, ((1,), (1,)))
            act = h * jax.nn.sigmoid(h) * v
            y = _dot(act, w3, ((1,), (1,)))
            out_ref[...] += _dot(p2, y, ((1,), (0,)))
            return 0

        lax.fori_loop(0, nblk_e, body, 0)


def kernel(x, Wg, W1, W2, W3):
    x2 = x.reshape(N, D)
    out = pl.pallas_call(
        _moe_kernel,
        in_specs=[
            pl.BlockSpec(memory_space=pltpu.VMEM),
            pl.BlockSpec(memory_space=pltpu.VMEM),
            pl.BlockSpec(memory_space=pl.ANY),
            pl.BlockSpec(memory_space=pl.ANY),
            pl.BlockSpec(memory_space=pl.ANY),
        ],
        out_shape=jax.ShapeDtypeStruct((N, D), _F32),
        scratch_shapes=[
            pltpu.VMEM((3, H, D), _F32),
            pltpu.VMEM((3, H, D), _F32),
            pltpu.VMEM((3, D, H), _F32),
            pltpu.SemaphoreType.DMA((3, 6)),
            pltpu.VMEM((2, E), _I32),
            pltpu.SMEM((2, E), _I32),
            pltpu.SemaphoreType.DMA,
        ],
        compiler_params=pltpu.CompilerParams(
            vmem_limit_bytes=100 * 1024 * 1024,
        ),
    )(x2, Wg, W1, W2, W3)
    return out.reshape(B, S, D)
